# Initial kernel scaffold; baseline (speedup 1.0000x reference)
import jax
import jax.numpy as jnp
from jax.experimental import pallas as pl


def kernel(logit, context, context_maps, context_bias, weights, bias, boolean_converter):
    raise NotImplementedError("write your pallas kernel here")


# speedup vs baseline: 7.3473x; 7.3473x over previous
import functools

import jax
import jax.numpy as jnp
import numpy as np
from jax.experimental import pallas as pl
from jax.experimental.pallas import tpu as pltpu

_LO = float(np.log(0.001 / 0.999))
_HI = -_LO


def _idx_body(cm_ref, cb_ref, ctx_ref, idx_ref, *, np_, m):
    d = jnp.dot(cm_ref[...], ctx_ref[...], preferred_element_type=jnp.float32)
    bits = (d > cb_ref[...]).astype(jnp.float32)
    acc = bits[0:np_]
    for j in range(1, m):
        acc = acc + float(2 ** j) * bits[j * np_:(j + 1) * np_]
    idx_ref[...] = acc


def _main_body(wt_ref, lg_ref, idx_ref, out_ref, *, nb):
    c = pl.program_id(0)
    pc = jnp.dot(wt_ref[0], lg_ref[...], preferred_element_type=jnp.float32)
    contrib = jnp.where(idx_ref[...] == c.astype(jnp.float32), pc, 0.0)

    @pl.when(c == 0)
    def _():
        out_ref[...] = contrib

    @pl.when(jnp.logical_and(c > 0, c < nb - 1))
    def _():
        out_ref[...] = out_ref[...] + contrib

    @pl.when(c == nb - 1)
    def _():
        out_ref[...] = jnp.clip(out_ref[...] + contrib, _LO, _HI)


def kernel(logit, context, context_maps, context_bias, weights, bias,
           boolean_converter):
    B, I = logit.shape
    _, C = context.shape
    K, S, M, _ = context_maps.shape
    NB = weights.shape[2]
    N = K * S
    NP = -(-N // 512) * 512

    cm = context_maps.reshape(N, M, C).transpose(1, 0, 2)
    cm = jnp.pad(cm, ((0, 0), (0, NP - N), (0, 0))).reshape(M * NP, C)
    cb = context_bias.reshape(N, M, 1).transpose(1, 0, 2)
    cb = jnp.pad(cb, ((0, 0), (0, NP - N), (0, 0))).reshape(M * NP, 1)
    ctxT = context.T

    idx = pl.pallas_call(
        functools.partial(_idx_body, np_=NP, m=M),
        out_shape=jax.ShapeDtypeStruct((NP, B), jnp.float32),
    )(cm, cb, ctxT)

    wt = weights.reshape(N, NB, I).transpose(1, 0, 2)
    wt = jnp.pad(wt, ((0, 0), (0, NP - N), (0, 0)))
    lgT = logit.T

    out = pl.pallas_call(
        functools.partial(_main_body, nb=NB),
        grid=(NB,),
        in_specs=[
            pl.BlockSpec((1, NP, I), lambda c: (c, 0, 0)),
            pl.BlockSpec((I, B), lambda c: (0, 0)),
            pl.BlockSpec((NP, B), lambda c: (0, 0)),
        ],
        out_specs=pl.BlockSpec((NP, B), lambda c: (0, 0)),
        out_shape=jax.ShapeDtypeStruct((NP, B), jnp.float32),
    )(wt, lgT, idx)

    body = out[:N].reshape(K, S, B).transpose(2, 1, 0)
    bias_append = jnp.broadcast_to(bias, (B, 1, K))
    return jnp.concatenate([bias_append, body], axis=1)


# speedup vs baseline: 9.9710x; 1.3571x over previous
import functools

import jax
import jax.numpy as jnp
import numpy as np
from jax.experimental import pallas as pl
from jax.experimental.pallas import tpu as pltpu

_LO = float(np.log(0.001 / 0.999))
_HI = -_LO


def _body(cm_ref, cb_ref, ctx_ref, w_ref, lg_ref, out_ref, *, sb, m, nb, n):
    nbit = sb * m
    nrow = sb * nb
    f32 = jnp.float32

    cm = cm_ref[...].reshape(nbit, cm_ref.shape[-1])
    d = jnp.dot(cm, ctx_ref[...], preferred_element_type=f32)
    cb = cb_ref[...].reshape(nbit, 1)
    bits = (d > cb).astype(f32)
    r = jax.lax.broadcasted_iota(jnp.int32, (sb, nbit), 1)
    s = jax.lax.broadcasted_iota(jnp.int32, (sb, nbit), 0)
    a4 = jnp.where(r // m == s,
                   jax.lax.shift_left(1, r % m).astype(f32), 0.0)
    idx = jnp.dot(a4, bits, preferred_element_type=f32)

    w = w_ref[...].reshape(nrow, w_ref.shape[-1])
    p = jnp.dot(w, lg_ref[...], preferred_element_type=f32)

    rr = jax.lax.broadcasted_iota(jnp.int32, (nrow, sb), 0)
    ss = jax.lax.broadcasted_iota(jnp.int32, (nrow, sb), 1)
    e = (rr // nb == ss).astype(f32)
    idx_exp = jax.lax.dot_general(e, idx, (((1,), (0,)), ((), ())),
                                  preferred_element_type=f32)
    riota = jax.lax.broadcasted_iota(jnp.int32, (nrow, idx.shape[-1]), 0)
    rbucket = (riota % nb).astype(f32)
    nv = n - pl.program_id(0) * sb
    valid = (riota // nb) < nv
    masked = jnp.where(jnp.logical_and(idx_exp == rbucket, valid), p, 0.0)
    acc = jax.lax.dot_general(e, masked, (((0,), (0,)), ((), ())),
                              preferred_element_type=f32)
    out_ref[...] = jnp.clip(acc, _LO, _HI)


def kernel(logit, context, context_maps, context_bias, weights, bias,
           boolean_converter):
    B, I = logit.shape
    _, C = context.shape
    K, S, M, _ = context_maps.shape
    NB = weights.shape[2]
    N = K * S
    SB = 32
    G = -(-N // SB)

    cm = context_maps.reshape(N, M, C)
    cb = context_bias.reshape(N, M, 1)
    wt = weights.reshape(N, NB, I)
    ctxT = context.T
    lgT = logit.T

    out = pl.pallas_call(
        functools.partial(_body, sb=SB, m=M, nb=NB, n=N),
        grid=(G,),
        in_specs=[
            pl.BlockSpec((SB, M, C), lambda i: (i, 0, 0)),
            pl.BlockSpec((SB, M, 1), lambda i: (i, 0, 0)),
            pl.BlockSpec((C, B), lambda i: (0, 0)),
            pl.BlockSpec((SB, NB, I), lambda i: (i, 0, 0)),
            pl.BlockSpec((I, B), lambda i: (0, 0)),
        ],
        out_specs=pl.BlockSpec((SB, B), lambda i: (i, 0)),
        out_shape=jax.ShapeDtypeStruct((N, B), jnp.float32),
    )(cm, cb, ctxT, wt, lgT)

    body = out.reshape(K, S, B).transpose(2, 1, 0)
    bias_append = jnp.broadcast_to(bias, (B, 1, K))
    return jnp.concatenate([bias_append, body], axis=1)
